# baseline (speedup 1.0000x reference)
import functools

import jax
import jax.numpy as jnp
from jax.experimental import pallas as pl
from jax.experimental.pallas import tpu as pltpu


def _round_up(n, m):
    return ((n + m - 1) // m) * m


_UG = 8


def _gather_body(pw_ref,
                 oh_ref,
                 wv_ref,
                 tab_ref,
                 out_ref,
                 *, tm, ws, d):
    i = pl.program_id(0)
    base = i * tm
    half = ws // 2

    out_ref[:, ws:d] = jnp.dot(oh_ref[...], tab_ref[...],
                               preferred_element_type=jnp.float32)

    sub = jax.lax.broadcasted_iota(jnp.int32, (8, 1), 0)

    def one_group(b, t0):
        acc = None
        for u in range(8):
            pw = pw_ref[b + u]
            w = wv_ref[pl.ds(pl.multiple_of(pw >> 3, 8), 8), :]
            wr = pltpu.roll(w, pw & 7, 0)
            acc = wr if u == 0 else jnp.where(sub == u, wr, acc)
        w0 = jax.lax.bitcast_convert_type(acc << 16, jnp.float32)
        w1 = jax.lax.bitcast_convert_type(acc & jnp.int32(-65536), jnp.float32)
        out_ref[pl.ds(t0, 8), 0:half] = w0
        out_ref[pl.ds(t0, 8), half:ws] = w1

    def block(g, carry):
        for j in range(_UG):
            t0 = pl.multiple_of(g * (8 * _UG) + 8 * j, 8)
            one_group(base + g * (8 * _UG) + 8 * j, t0)
        return carry

    jax.lax.fori_loop(0, tm // (8 * _UG), block, 0)


@jax.jit
def kernel(x, ldist, rdist, Wv, pf1, pf2):
    B, S = x.shape
    WL, WS = Wv.shape
    FL, FS = pf1.shape
    FL2, FS2 = pf2.shape
    LR = FS + FS2
    D = WS + LR
    N = B * S

    xi = jnp.clip(x.reshape(N).astype(jnp.int32), 0, WL - 1)
    li = jnp.clip(ldist.reshape(N).astype(jnp.int32), 0, FL - 1)
    ri = jnp.clip(rdist.reshape(N).astype(jnp.int32), 0, FL2 - 1)

    tm = min(2048, _round_up(N, 8))
    n_pad = _round_up(N, tm)
    pad = n_pad - N
    if pad:
        zero = jnp.zeros((pad,), jnp.int32)
        xi = jnp.concatenate([xi, zero])
        li = jnp.concatenate([li, zero])
        ri = jnp.concatenate([ri, zero])

    tpos = jax.lax.iota(jnp.int32, n_pad) & 7
    pw = ((xi & ~7) << 3) | ((tpos - xi) & 7)
    vv = jax.lax.iota(jnp.int32, FL + FL2)
    oh = ((li[:, None] == vv[None, :])
          | ((ri + FL)[:, None] == vv[None, :])).astype(jnp.float32)

    wv_bf = Wv.astype(jnp.bfloat16)
    wv_pack = jax.lax.bitcast_convert_type(
        wv_bf.reshape(WL, 2, WS // 2).transpose(0, 2, 1), jnp.int32)

    tab = jnp.zeros((FL + FL2, LR), jnp.float32)
    tab = tab.at[:FL, :FS].set(pf1.astype(jnp.float32))
    tab = tab.at[FL:, FS:].set(pf2.astype(jnp.float32))

    vmem_bytes = (WL * 128 * 4 + (FL + FL2) * LR * 4 + 2 * tm * D * 4
                  + 2 * tm * (FL + FL2) * 4 + (1 << 20))
    out = pl.pallas_call(
        functools.partial(_gather_body, tm=tm, ws=WS, d=D),
        out_shape=jax.ShapeDtypeStruct((n_pad, D), jnp.float32),
        grid_spec=pltpu.PrefetchScalarGridSpec(
            num_scalar_prefetch=1,
            grid=(n_pad // tm,),
            in_specs=[
                pl.BlockSpec((tm, FL + FL2), lambda i, *_: (i, 0)),
                pl.BlockSpec((WL, 128), lambda i, *_: (0, 0)),
                pl.BlockSpec((FL + FL2, LR), lambda i, *_: (0, 0)),
            ],
            out_specs=pl.BlockSpec((tm, D), lambda i, *_: (i, 0)),
        ),
        compiler_params=pltpu.CompilerParams(
            dimension_semantics=("arbitrary",),
            vmem_limit_bytes=min(vmem_bytes, 60 * 1024 * 1024),
        ),
    )(pw, oh, wv_pack, tab)

    return out[:N].reshape(B, S, D)[:, None, :, :]


# speedup vs baseline: 2.4203x; 2.4203x over previous
import functools

import jax
import jax.numpy as jnp
from jax.experimental import pallas as pl
from jax.experimental.pallas import tpu as pltpu


def _round_up(n, m):
    return ((n + m - 1) // m) * m


_UG = 8


def _gather_body(pw_ref,
                 oh_ref,
                 wv_ref,
                 tab_ref,
                 out_ref,
                 *, tm, ws, d):
    i = pl.program_id(0)
    base = i * tm
    half = ws // 2

    out_ref[:, ws:d] = jnp.dot(oh_ref[...], tab_ref[...],
                               preferred_element_type=jnp.float32)

    sub = jax.lax.broadcasted_iota(jnp.int32, (8, 1), 0)

    def one_group(b, t0):
        acc = None
        for u in range(8):
            pw = pw_ref[b + u]
            w = wv_ref[pl.ds(pl.multiple_of(pw >> 3, 8), 8), :]
            wr = pltpu.roll(w, pw & 7, 0)
            acc = wr if u == 0 else jnp.where(sub == u, wr, acc)
        w0 = jax.lax.bitcast_convert_type(acc << 16, jnp.float32)
        w1 = jax.lax.bitcast_convert_type(acc & jnp.int32(-65536), jnp.float32)
        out_ref[pl.ds(t0, 8), 0:half] = w0
        out_ref[pl.ds(t0, 8), half:ws] = w1

    def block(g, carry):
        for j in range(_UG):
            t0 = pl.multiple_of(g * (8 * _UG) + 8 * j, 8)
            one_group(base + g * (8 * _UG) + 8 * j, t0)
        return carry

    jax.lax.fori_loop(0, tm // (8 * _UG), block, 0)


@jax.jit
def kernel(x, ldist, rdist, Wv, pf1, pf2):
    B, S = x.shape
    WL, WS = Wv.shape
    FL, FS = pf1.shape
    FL2, FS2 = pf2.shape
    LR = FS + FS2
    D = WS + LR
    N = B * S

    xi = jnp.clip(x.reshape(N).astype(jnp.int32), 0, WL - 1)
    li = jnp.clip(ldist.reshape(N).astype(jnp.int32), 0, FL - 1)
    ri = jnp.clip(rdist.reshape(N).astype(jnp.int32), 0, FL2 - 1)

    tm = min(2048, _round_up(N, 8))
    n_pad = _round_up(N, tm)
    pad = n_pad - N
    if pad:
        zero = jnp.zeros((pad,), jnp.int32)
        xi = jnp.concatenate([xi, zero])
        li = jnp.concatenate([li, zero])
        ri = jnp.concatenate([ri, zero])

    tpos = jax.lax.iota(jnp.int32, n_pad) & 7
    pw = ((xi & ~7) << 3) | ((tpos - xi) & 7)
    vv = jax.lax.iota(jnp.int32, FL + FL2)
    oh = ((li[:, None] == vv[None, :])
          | ((ri + FL)[:, None] == vv[None, :])).astype(jnp.float32)

    vi = jax.lax.bitcast_convert_type(Wv, jnp.int32)
    vb = (vi + 0x7FFF + ((vi >> 16) & 1)) >> 16
    wv_pack = (vb[:, :WS // 2] & 0xFFFF) | (vb[:, WS // 2:] << 16)

    tab = jnp.zeros((FL + FL2, LR), jnp.float32)
    tab = tab.at[:FL, :FS].set(pf1.astype(jnp.float32))
    tab = tab.at[FL:, FS:].set(pf2.astype(jnp.float32))

    vmem_bytes = (WL * 128 * 4 + (FL + FL2) * LR * 4 + 2 * tm * D * 4
                  + 2 * tm * (FL + FL2) * 4 + (1 << 20))
    out = pl.pallas_call(
        functools.partial(_gather_body, tm=tm, ws=WS, d=D),
        out_shape=jax.ShapeDtypeStruct((n_pad, D), jnp.float32),
        grid_spec=pltpu.PrefetchScalarGridSpec(
            num_scalar_prefetch=1,
            grid=(n_pad // tm,),
            in_specs=[
                pl.BlockSpec((tm, FL + FL2), lambda i, *_: (i, 0)),
                pl.BlockSpec((WL, 128), lambda i, *_: (0, 0)),
                pl.BlockSpec((FL + FL2, LR), lambda i, *_: (0, 0)),
            ],
            out_specs=pl.BlockSpec((tm, D), lambda i, *_: (i, 0)),
        ),
        compiler_params=pltpu.CompilerParams(
            dimension_semantics=("arbitrary",),
            vmem_limit_bytes=min(vmem_bytes, 60 * 1024 * 1024),
        ),
    )(pw, oh, wv_pack, tab)

    return out[:N].reshape(B, S, D)[:, None, :, :]


# speedup vs baseline: 2.9477x; 1.2179x over previous
import functools

import jax
import jax.numpy as jnp
from jax.experimental import pallas as pl
from jax.experimental.pallas import tpu as pltpu


def _round_up(n, m):
    return ((n + m - 1) // m) * m


_UG = 16


def _gather_body(pw_ref,
                 oh_ref,
                 wv_ref,
                 tab_ref,
                 out_ref,
                 *, tm, ws, d):
    i = pl.program_id(0)
    base = i * tm

    out_ref[:, ws:d] = jnp.dot(oh_ref[...], tab_ref[...],
                               preferred_element_type=jnp.float32)

    sub = jax.lax.broadcasted_iota(jnp.int32, (8, 1), 0)

    def one_group(b, t0):
        accw = None
        for u in range(8):
            pw = pw_ref[b + u]
            w = wv_ref[pl.ds(pl.multiple_of(pw >> 3, 8), 8), :]
            wr = pltpu.roll(w, pw & 7, 0)
            accw = wr if u == 0 else jnp.where(sub == u, wr, accw)
        out_ref[pl.ds(t0, 8), 0:ws] = accw

    def block(g, carry):
        for j in range(_UG):
            t0 = pl.multiple_of(g * (8 * _UG) + 8 * j, 8)
            one_group(base + g * (8 * _UG) + 8 * j, t0)
        return carry

    jax.lax.fori_loop(0, tm // (8 * _UG), block, 0)


@jax.jit
def kernel(x, ldist, rdist, Wv, pf1, pf2):
    B, S = x.shape
    WL, WS = Wv.shape
    FL, FS = pf1.shape
    FL2, FS2 = pf2.shape
    LR = FS + FS2
    D = WS + LR
    N = B * S

    xi = jnp.clip(x.reshape(N).astype(jnp.int32), 0, WL - 1)
    li = jnp.clip(ldist.reshape(N).astype(jnp.int32), 0, FL - 1)
    ri = jnp.clip(rdist.reshape(N).astype(jnp.int32), 0, FL2 - 1)

    tm = min(2048, _round_up(N, 8))
    n_pad = _round_up(N, tm)
    pad = n_pad - N
    if pad:
        zero = jnp.zeros((pad,), jnp.int32)
        xi = jnp.concatenate([xi, zero])
        li = jnp.concatenate([li, zero])
        ri = jnp.concatenate([ri, zero])

    tpos = jax.lax.iota(jnp.int32, n_pad) & 7
    pw = ((xi & ~7) << 3) | ((tpos - xi) & 7)
    vv = jax.lax.iota(jnp.int32, FL + FL2)
    oh = ((li[:, None] == vv[None, :])
          | ((ri + FL)[:, None] == vv[None, :])).astype(jnp.float32)

    tab = jnp.zeros((FL + FL2, LR), jnp.float32)
    tab = tab.at[:FL, :FS].set(pf1.astype(jnp.float32))
    tab = tab.at[FL:, FS:].set(pf2.astype(jnp.float32))

    vmem_bytes = (WL * WS * 4 + (FL + FL2) * LR * 4 + 2 * tm * D * 4
                  + 2 * tm * 128 * 4 + tm * (FL + FL2) * 4 + (1 << 20))
    out = pl.pallas_call(
        functools.partial(_gather_body, tm=tm, ws=WS, d=D),
        out_shape=jax.ShapeDtypeStruct((n_pad, D), jnp.float32),
        grid_spec=pltpu.PrefetchScalarGridSpec(
            num_scalar_prefetch=1,
            grid=(n_pad // tm,),
            in_specs=[
                pl.BlockSpec((tm, FL + FL2), lambda i, *_: (i, 0)),
                pl.BlockSpec((WL, WS), lambda i, *_: (0, 0)),
                pl.BlockSpec((FL + FL2, LR), lambda i, *_: (0, 0)),
            ],
            out_specs=pl.BlockSpec((tm, D), lambda i, *_: (i, 0)),
        ),
        compiler_params=pltpu.CompilerParams(
            dimension_semantics=("arbitrary",),
            vmem_limit_bytes=min(vmem_bytes, 60 * 1024 * 1024),
        ),
    )(pw, oh, Wv, tab)

    return out[:N].reshape(B, S, D)[:, None, :, :]


# speedup vs baseline: 2.9490x; 1.0005x over previous
import functools

import jax
import jax.numpy as jnp
from jax.experimental import pallas as pl
from jax.experimental.pallas import tpu as pltpu


def _round_up(n, m):
    return ((n + m - 1) // m) * m


_UG = 16


def _gather_body(pw_ref,
                 oh_ref,
                 wv_ref,
                 tab_ref,
                 out_ref,
                 *, tm, ws, d):
    i = pl.program_id(0)
    base = i * tm

    out_ref[:, ws:d] = jnp.dot(oh_ref[...], tab_ref[...],
                               preferred_element_type=jnp.float32)

    sub = jax.lax.broadcasted_iota(jnp.int32, (8, 1), 0)

    def one_group(b, t0):
        accw = None
        for u in range(8):
            pw = pw_ref[b + u]
            w = wv_ref[pl.ds(pl.multiple_of(pw >> 3, 8), 8), :]
            wr = pltpu.roll(w, pw & 7, 0)
            accw = wr if u == 0 else jnp.where(sub == u, wr, accw)
        out_ref[pl.ds(t0, 8), 0:ws] = accw

    def block(g, carry):
        for j in range(_UG):
            t0 = pl.multiple_of(g * (8 * _UG) + 8 * j, 8)
            one_group(base + g * (8 * _UG) + 8 * j, t0)
        return carry

    jax.lax.fori_loop(0, tm // (8 * _UG), block, 0)


@jax.jit
def kernel(x, ldist, rdist, Wv, pf1, pf2):
    B, S = x.shape
    WL, WS = Wv.shape
    FL, FS = pf1.shape
    FL2, FS2 = pf2.shape
    LR = FS + FS2
    D = WS + LR
    N = B * S

    xi = jnp.clip(x.reshape(N).astype(jnp.int32), 0, WL - 1)
    li = jnp.clip(ldist.reshape(N).astype(jnp.int32), 0, FL - 1)
    ri = jnp.clip(rdist.reshape(N).astype(jnp.int32), 0, FL2 - 1)

    tm = min(4096, _round_up(N, 8))
    n_pad = _round_up(N, tm)
    pad = n_pad - N
    if pad:
        zero = jnp.zeros((pad,), jnp.int32)
        xi = jnp.concatenate([xi, zero])
        li = jnp.concatenate([li, zero])
        ri = jnp.concatenate([ri, zero])

    tpos = jax.lax.iota(jnp.int32, n_pad) & 7
    pw = ((xi & ~7) << 3) | ((tpos - xi) & 7)
    vv = jax.lax.iota(jnp.int32, FL + FL2)
    oh = ((li[:, None] == vv[None, :])
          | ((ri + FL)[:, None] == vv[None, :])).astype(jnp.float32)

    tab = jnp.zeros((FL + FL2, LR), jnp.float32)
    tab = tab.at[:FL, :FS].set(pf1.astype(jnp.float32))
    tab = tab.at[FL:, FS:].set(pf2.astype(jnp.float32))

    vmem_bytes = (WL * WS * 4 + (FL + FL2) * LR * 4 + 2 * tm * D * 4
                  + 2 * tm * 128 * 4 + tm * (FL + FL2) * 4 + (1 << 20))
    out = pl.pallas_call(
        functools.partial(_gather_body, tm=tm, ws=WS, d=D),
        out_shape=jax.ShapeDtypeStruct((n_pad, D), jnp.float32),
        grid_spec=pltpu.PrefetchScalarGridSpec(
            num_scalar_prefetch=1,
            grid=(n_pad // tm,),
            in_specs=[
                pl.BlockSpec((tm, FL + FL2), lambda i, *_: (i, 0)),
                pl.BlockSpec((WL, WS), lambda i, *_: (0, 0)),
                pl.BlockSpec((FL + FL2, LR), lambda i, *_: (0, 0)),
            ],
            out_specs=pl.BlockSpec((tm, D), lambda i, *_: (i, 0)),
        ),
        compiler_params=pltpu.CompilerParams(
            dimension_semantics=("arbitrary",),
            vmem_limit_bytes=min(vmem_bytes, 60 * 1024 * 1024),
        ),
    )(pw, oh, Wv, tab)

    return out[:N].reshape(B, S, D)[:, None, :, :]


# speedup vs baseline: 2.9809x; 1.0108x over previous
import functools

import jax
import jax.numpy as jnp
from jax.experimental import pallas as pl
from jax.experimental.pallas import tpu as pltpu


def _round_up(n, m):
    return ((n + m - 1) // m) * m


_UG = 16


def _gather_body(pw_ref,
                 oh_ref,
                 wv_ref,
                 tab_ref,
                 out_ref,
                 *, tm, ws, d):
    i = pl.program_id(0)
    base = i * tm

    out_ref[:, ws:d] = jnp.dot(oh_ref[...], tab_ref[...],
                               preferred_element_type=jnp.float32)

    sub = jax.lax.broadcasted_iota(jnp.int32, (8, 1), 0)

    def one_group(b, t0):
        accw = None
        for u in range(8):
            pw = pw_ref[b + u]
            w = wv_ref[pl.ds(pl.multiple_of(pw >> 3, 8), 8), :]
            wr = pltpu.roll(w, pw & 7, 0)
            accw = wr if u == 0 else jnp.where(sub == u, wr, accw)
        out_ref[pl.ds(t0, 8), 0:ws] = accw

    def block(g, carry):
        for j in range(_UG):
            t0 = pl.multiple_of(g * (8 * _UG) + 8 * j, 8)
            one_group(base + g * (8 * _UG) + 8 * j, t0)
        return carry

    jax.lax.fori_loop(0, tm // (8 * _UG), block, 0)


@jax.jit
def kernel(x, ldist, rdist, Wv, pf1, pf2):
    B, S = x.shape
    WL, WS = Wv.shape
    FL, FS = pf1.shape
    FL2, FS2 = pf2.shape
    LR = FS + FS2
    D = WS + LR
    N = B * S

    xi = jnp.clip(x.reshape(N).astype(jnp.int32), 0, WL - 1)
    li = jnp.clip(ldist.reshape(N).astype(jnp.int32), 0, FL - 1)
    ri = jnp.clip(rdist.reshape(N).astype(jnp.int32), 0, FL2 - 1)

    tm = min(4096, _round_up(N, 8))
    n_pad = _round_up(N, tm)
    pad = n_pad - N
    if pad:
        zero = jnp.zeros((pad,), jnp.int32)
        xi = jnp.concatenate([xi, zero])
        li = jnp.concatenate([li, zero])
        ri = jnp.concatenate([ri, zero])

    tpos = jax.lax.iota(jnp.int32, n_pad) & 7
    pw = ((xi & ~7) << 3) | ((tpos - xi) & 7)
    vv = jax.lax.iota(jnp.int32, FL + FL2)
    oh = ((li[:, None] == vv[None, :])
          | ((ri + FL)[:, None] == vv[None, :])).astype(jnp.bfloat16)

    tab = jnp.zeros((FL + FL2, LR), jnp.bfloat16)
    tab = tab.at[:FL, :FS].set(pf1.astype(jnp.bfloat16))
    tab = tab.at[FL:, FS:].set(pf2.astype(jnp.bfloat16))

    vmem_bytes = (WL * WS * 4 + (FL + FL2) * LR * 4 + 2 * tm * D * 4
                  + 2 * tm * 128 * 4 + tm * (FL + FL2) * 4 + (1 << 20))
    out = pl.pallas_call(
        functools.partial(_gather_body, tm=tm, ws=WS, d=D),
        out_shape=jax.ShapeDtypeStruct((n_pad, D), jnp.float32),
        grid_spec=pltpu.PrefetchScalarGridSpec(
            num_scalar_prefetch=1,
            grid=(n_pad // tm,),
            in_specs=[
                pl.BlockSpec((tm, FL + FL2), lambda i, *_: (i, 0)),
                pl.BlockSpec((WL, WS), lambda i, *_: (0, 0)),
                pl.BlockSpec((FL + FL2, LR), lambda i, *_: (0, 0)),
            ],
            out_specs=pl.BlockSpec((tm, D), lambda i, *_: (i, 0)),
        ),
        compiler_params=pltpu.CompilerParams(
            dimension_semantics=("arbitrary",),
            vmem_limit_bytes=min(vmem_bytes, 60 * 1024 * 1024),
        ),
    )(pw, oh, Wv, tab)

    return out[:N].reshape(B, S, D)[:, None, :, :]
